# Initial kernel scaffold; baseline (speedup 1.0000x reference)
import jax
import jax.numpy as jnp
from jax.experimental import pallas as pl


def kernel(src, begin_offsets, end_offsets, wq, wk, wv, bq, bk, bv, wo, bo, w1, b1, w2, b2, g1, be1, g2, be2):
    raise NotImplementedError("write your pallas kernel here")


# speedup vs baseline: 4.7332x; 4.7332x over previous
import math

import jax
import jax.numpy as jnp
from jax.experimental import pallas as pl
from jax.experimental.pallas import tpu as pltpu

L = 128
D_MODEL = 32
NHEAD = 4
HEAD_DIM = D_MODEL // NHEAD
DFF = 64
N_LAYERS = 2
EPS = 1e-5
NEG_INF = -1e30
BBLK = 16


def _ln(x, g, b):
    mu = jnp.mean(x, axis=-1, keepdims=True)
    var = jnp.mean((x - mu) ** 2, axis=-1, keepdims=True)
    return (x - mu) * jax.lax.rsqrt(var + EPS) * g + b


def _encoder_kernel(x_ref, bias_ref, wqkv_ref, wo_ref, w1_ref, w2_ref,
                    bqkv_ref, b1_ref, vec_ref, y_ref, attn_ref):
    bblk = x_ref.shape[1]
    M = bblk * L

    rh = jax.lax.broadcasted_iota(jnp.int32, (NHEAD * L, D_MODEL), 0) // L
    ch = jax.lax.broadcasted_iota(jnp.int32, (NHEAD * L, D_MODEL), 1) // HEAD_DIM
    headmask = (rh == ch).astype(jnp.bfloat16)

    xm = jnp.transpose(x_ref[...], (1, 0, 2)).reshape(M, D_MODEL)
    bias = bias_ref[...]

    for layer in range(N_LAYERS):
        vec = vec_ref[layer]
        bo, b2 = vec[0:1], vec[1:2]
        g1, be1 = vec[2:3], vec[3:4]
        g2, be2 = vec[4:5], vec[5:6]

        qkv = jnp.dot(xm.astype(jnp.bfloat16), wqkv_ref[layer],
                      preferred_element_type=jnp.float32) + bqkv_ref[layer]
        q = qkv[:, :D_MODEL].astype(jnp.bfloat16).reshape(bblk, L, D_MODEL)
        k = qkv[:, D_MODEL:2 * D_MODEL].astype(jnp.bfloat16)
        v = qkv[:, 2 * D_MODEL:].astype(jnp.bfloat16)
        k = k.reshape(bblk, L, D_MODEL)
        v = v.reshape(bblk, L, D_MODEL)

        k4 = jnp.concatenate([k, k, k, k], axis=1) * headmask
        v4 = jnp.concatenate([v, v, v, v], axis=1) * headmask

        s = jax.lax.dot_general(q, k4, (((2,), (2,)), ((0,), (0,))),
                                preferred_element_type=jnp.float32)

        pn_parts = []
        acc = None
        for h in range(NHEAD):
            sh = s[:, :, h * L:(h + 1) * L] + bias[:, None, :]
            m = jnp.max(sh, axis=-1, keepdims=True)
            p = jnp.exp(sh - m)
            inv = 1.0 / jnp.sum(p, axis=-1, keepdims=True)
            pnh = p * inv
            acc = pnh if acc is None else acc + pnh
            pn_parts.append(pnh.astype(jnp.bfloat16))
        attn_ref[layer] = acc * (1.0 / NHEAD)
        pn = jnp.concatenate(pn_parts, axis=2)

        ctx = jax.lax.dot_general(pn, v4, (((2,), (1,)), ((0,), (0,))),
                                  preferred_element_type=jnp.float32)
        attn_out = jnp.dot(ctx.reshape(M, D_MODEL).astype(jnp.bfloat16),
                           wo_ref[layer], preferred_element_type=jnp.float32)

        h1 = _ln(xm + attn_out + bo, g1, be1)
        f = jnp.dot(h1.astype(jnp.bfloat16), w1_ref[layer],
                    preferred_element_type=jnp.float32) + b1_ref[layer]
        f = jnp.maximum(f, 0.0)
        f = jnp.dot(f.astype(jnp.bfloat16), w2_ref[layer],
                    preferred_element_type=jnp.float32) + b2
        xm = _ln(h1 + f, g2, be2)

    y_ref[...] = jnp.transpose(xm.reshape(bblk, L, D_MODEL), (1, 0, 2))


def kernel(src, begin_offsets, end_offsets, wq, wk, wv, bq, bk, bv,
           wo, bo, w1, b1, w2, b2, g1, be1, g2, be2):
    del begin_offsets
    B = src.shape[1]
    nb = pl.cdiv(B, BBLK)
    scale = 1.0 / math.sqrt(HEAD_DIM)

    wqkv = jnp.concatenate([wq * scale, wk, wv], axis=2).astype(jnp.bfloat16)
    bqkv = jnp.concatenate([bq * scale, bk, bv], axis=2)
    vec = jnp.concatenate([bo, b2, g1, be1, g2, be2], axis=1)

    bias = jnp.where(jnp.arange(L)[None, :] >= end_offsets[:, None],
                     NEG_INF, 0.0).astype(jnp.float32)

    in_specs = [
        pl.BlockSpec((L, BBLK, D_MODEL), lambda b: (0, b, 0)),
        pl.BlockSpec((BBLK, L), lambda b: (b, 0)),
        pl.BlockSpec((N_LAYERS, D_MODEL, 3 * D_MODEL), lambda b: (0, 0, 0)),
        pl.BlockSpec((N_LAYERS, D_MODEL, D_MODEL), lambda b: (0, 0, 0)),
        pl.BlockSpec((N_LAYERS, D_MODEL, DFF), lambda b: (0, 0, 0)),
        pl.BlockSpec((N_LAYERS, DFF, D_MODEL), lambda b: (0, 0, 0)),
        pl.BlockSpec((N_LAYERS, 1, 3 * D_MODEL), lambda b: (0, 0, 0)),
        pl.BlockSpec((N_LAYERS, 1, DFF), lambda b: (0, 0, 0)),
        pl.BlockSpec((N_LAYERS, 6, D_MODEL), lambda b: (0, 0, 0)),
    ]
    out_specs = [
        pl.BlockSpec((L, BBLK, D_MODEL), lambda b: (0, b, 0)),
        pl.BlockSpec((N_LAYERS, BBLK, L, L), lambda b: (0, b, 0, 0)),
    ]
    out_shape = [
        jax.ShapeDtypeStruct((L, B, D_MODEL), jnp.float32),
        jax.ShapeDtypeStruct((N_LAYERS, B, L, L), jnp.float32),
    ]

    y, attn = pl.pallas_call(
        _encoder_kernel,
        grid=(nb,),
        in_specs=in_specs,
        out_specs=out_specs,
        out_shape=out_shape,
        compiler_params=pltpu.CompilerParams(
            dimension_semantics=("parallel",)),
    )(src, bias, wqkv, wo.astype(jnp.bfloat16),
      w1.astype(jnp.bfloat16), w2.astype(jnp.bfloat16), bqkv, b1, vec)
    return y, attn


# speedup vs baseline: 5.0430x; 1.0654x over previous
import math

import jax
import jax.numpy as jnp
from jax.experimental import pallas as pl
from jax.experimental.pallas import tpu as pltpu

L = 128
D_MODEL = 32
NHEAD = 4
HEAD_DIM = D_MODEL // NHEAD
DFF = 64
N_LAYERS = 2
EPS = 1e-5
NEG_INF = -1e30
BBLK = 16


def _ln(x, g, b):
    mu = jnp.mean(x, axis=-1, keepdims=True)
    var = jnp.mean((x - mu) ** 2, axis=-1, keepdims=True)
    return (x - mu) * jax.lax.rsqrt(var + EPS) * g + b


def _encoder_kernel(x_ref, bias_ref, hm_ref, wqkv_ref, wo_ref, w1_ref, w2_ref,
                    bqkv_ref, b1_ref, vec_ref, y_ref, attn_ref):
    bblk = x_ref.shape[1]
    M = bblk * L

    headmask = hm_ref[...]

    xm = jnp.transpose(x_ref[...], (1, 0, 2)).reshape(M, D_MODEL)
    bias = bias_ref[...]

    for layer in range(N_LAYERS):
        vec = vec_ref[layer]
        bo, b2 = vec[0:1], vec[1:2]
        g1, be1 = vec[2:3], vec[3:4]
        g2, be2 = vec[4:5], vec[5:6]

        xb = xm.astype(jnp.bfloat16)
        wqkv = wqkv_ref[layer]
        bqkv = bqkv_ref[layer]
        q = (jnp.dot(xb, wqkv[:, :D_MODEL],
                     preferred_element_type=jnp.float32)
             + bqkv[:, :D_MODEL]).astype(jnp.bfloat16).reshape(
                 bblk, L, D_MODEL)
        k = (jnp.dot(xb, wqkv[:, D_MODEL:2 * D_MODEL],
                     preferred_element_type=jnp.float32)
             + bqkv[:, D_MODEL:2 * D_MODEL]).astype(jnp.bfloat16).reshape(
                 bblk, L, D_MODEL)
        v = (jnp.dot(xb, wqkv[:, 2 * D_MODEL:],
                     preferred_element_type=jnp.float32)
             + bqkv[:, 2 * D_MODEL:]).astype(jnp.bfloat16).reshape(
                 bblk, L, D_MODEL)

        k4 = jnp.concatenate([k, k, k, k], axis=1) * headmask
        v4 = jnp.concatenate([v, v, v, v], axis=1) * headmask

        s = jax.lax.dot_general(q, k4, (((2,), (2,)), ((0,), (0,))),
                                preferred_element_type=jnp.float32)

        pn_parts = []
        pf_parts = []
        for h in range(NHEAD):
            sh = s[:, :, h * L:(h + 1) * L] + bias[:, None, :]
            m = jnp.max(sh, axis=-1, keepdims=True)
            p = jnp.exp2(sh - m)
            inv = 1.0 / jnp.sum(p, axis=-1, keepdims=True)
            pnh = p * inv
            pf_parts.append(pnh)
            pn_parts.append(pnh.astype(jnp.bfloat16))
        attn_ref[layer] = ((pf_parts[0] + pf_parts[1])
                           + (pf_parts[2] + pf_parts[3])) * (1.0 / NHEAD)
        pn = jnp.concatenate(pn_parts, axis=2)

        ctx = jax.lax.dot_general(pn, v4, (((2,), (1,)), ((0,), (0,))),
                                  preferred_element_type=jnp.float32)
        attn_out = jnp.dot(ctx.reshape(M, D_MODEL).astype(jnp.bfloat16),
                           wo_ref[layer], preferred_element_type=jnp.float32)

        h1 = _ln(xm + attn_out + bo, g1, be1)
        f = jnp.dot(h1.astype(jnp.bfloat16), w1_ref[layer],
                    preferred_element_type=jnp.float32) + b1_ref[layer]
        f = jnp.maximum(f, 0.0)
        f = jnp.dot(f.astype(jnp.bfloat16), w2_ref[layer],
                    preferred_element_type=jnp.float32) + b2
        xm = _ln(h1 + f, g2, be2)

    y_ref[...] = jnp.transpose(xm.reshape(bblk, L, D_MODEL), (1, 0, 2))


def kernel(src, begin_offsets, end_offsets, wq, wk, wv, bq, bk, bv,
           wo, bo, w1, b1, w2, b2, g1, be1, g2, be2):
    del begin_offsets
    B = src.shape[1]
    nb = pl.cdiv(B, BBLK)
    scale = math.log2(math.e) / math.sqrt(HEAD_DIM)

    rows = jnp.arange(NHEAD * L)[:, None] // L
    cols = jnp.arange(D_MODEL)[None, :] // HEAD_DIM
    headmask = (rows == cols).astype(jnp.bfloat16)

    wqkv = jnp.concatenate([wq * scale, wk, wv], axis=2).astype(jnp.bfloat16)
    bqkv = jnp.concatenate([bq * scale, bk, bv], axis=2)
    vec = jnp.concatenate([bo, b2, g1, be1, g2, be2], axis=1)

    bias = jnp.where(jnp.arange(L)[None, :] >= end_offsets[:, None],
                     NEG_INF, 0.0).astype(jnp.float32)

    in_specs = [
        pl.BlockSpec((L, BBLK, D_MODEL), lambda b: (0, b, 0)),
        pl.BlockSpec((BBLK, L), lambda b: (b, 0)),
        pl.BlockSpec((NHEAD * L, D_MODEL), lambda b: (0, 0)),
        pl.BlockSpec((N_LAYERS, D_MODEL, 3 * D_MODEL), lambda b: (0, 0, 0)),
        pl.BlockSpec((N_LAYERS, D_MODEL, D_MODEL), lambda b: (0, 0, 0)),
        pl.BlockSpec((N_LAYERS, D_MODEL, DFF), lambda b: (0, 0, 0)),
        pl.BlockSpec((N_LAYERS, DFF, D_MODEL), lambda b: (0, 0, 0)),
        pl.BlockSpec((N_LAYERS, 1, 3 * D_MODEL), lambda b: (0, 0, 0)),
        pl.BlockSpec((N_LAYERS, 1, DFF), lambda b: (0, 0, 0)),
        pl.BlockSpec((N_LAYERS, 6, D_MODEL), lambda b: (0, 0, 0)),
    ]
    out_specs = [
        pl.BlockSpec((L, BBLK, D_MODEL), lambda b: (0, b, 0)),
        pl.BlockSpec((N_LAYERS, BBLK, L, L), lambda b: (0, b, 0, 0)),
    ]
    out_shape = [
        jax.ShapeDtypeStruct((L, B, D_MODEL), jnp.float32),
        jax.ShapeDtypeStruct((N_LAYERS, B, L, L), jnp.float32),
    ]

    y, attn = pl.pallas_call(
        _encoder_kernel,
        grid=(nb,),
        in_specs=in_specs,
        out_specs=out_specs,
        out_shape=out_shape,
        compiler_params=pltpu.CompilerParams(
            dimension_semantics=("parallel",)),
    )(src, bias, headmask, wqkv, wo.astype(jnp.bfloat16),
      w1.astype(jnp.bfloat16), w2.astype(jnp.bfloat16), bqkv, b1, vec)
    return y, attn


# speedup vs baseline: 5.2154x; 1.0342x over previous
import math

import jax
import jax.numpy as jnp
from jax.experimental import pallas as pl
from jax.experimental.pallas import tpu as pltpu

L = 128
D_MODEL = 32
NHEAD = 4
HEAD_DIM = D_MODEL // NHEAD
DFF = 64
N_LAYERS = 2
EPS = 1e-5
NEG_INF = -1e30
BBLK = 16


def _ln(x, g, b):
    mu = jnp.mean(x, axis=-1, keepdims=True)
    var = jnp.mean((x - mu) ** 2, axis=-1, keepdims=True)
    return (x - mu) * jax.lax.rsqrt(var + EPS) * g + b


def _encoder_kernel(x_ref, bias_ref, hm_ref, wqkv_ref, wo_ref, w1_ref, w2_ref,
                    bqkv_ref, b1_ref, vec_ref, y_ref, attn_ref):
    bblk = x_ref.shape[1]
    M = bblk * L

    headmask = hm_ref[...]

    xm = jnp.transpose(x_ref[...], (1, 0, 2)).reshape(M, D_MODEL)
    bias = bias_ref[...]

    for layer in range(N_LAYERS):
        vec = vec_ref[layer]
        bo, b2 = vec[0:1], vec[1:2]
        g1, be1 = vec[2:3], vec[3:4]
        g2, be2 = vec[4:5], vec[5:6]

        xb = xm.astype(jnp.bfloat16)
        wqkv = wqkv_ref[layer]
        bqkv = bqkv_ref[layer]
        q = (jnp.dot(xb, wqkv[:, :D_MODEL],
                     preferred_element_type=jnp.float32)
             + bqkv[:, :D_MODEL]).astype(jnp.bfloat16).reshape(
                 bblk, L, D_MODEL)
        k = (jnp.dot(xb, wqkv[:, D_MODEL:2 * D_MODEL],
                     preferred_element_type=jnp.float32)
             + bqkv[:, D_MODEL:2 * D_MODEL]).astype(jnp.bfloat16).reshape(
                 bblk, L, D_MODEL)
        v = (jnp.dot(xb, wqkv[:, 2 * D_MODEL:],
                     preferred_element_type=jnp.float32)
             + bqkv[:, 2 * D_MODEL:]).astype(jnp.bfloat16).reshape(
                 bblk, L, D_MODEL)

        k4 = jnp.concatenate([k, k, k, k], axis=1) * headmask
        v4 = jnp.concatenate([v, v, v, v], axis=1) * headmask

        s = jax.lax.dot_general(q, k4, (((2,), (2,)), ((0,), (0,))),
                                preferred_element_type=jnp.float32)

        pn_parts = []
        pf_parts = []
        for h in range(NHEAD):
            sh = s[:, :, h * L:(h + 1) * L] + bias[:, None, :]
            m = jnp.max(sh, axis=-1, keepdims=True)
            p = jnp.exp2(sh - m)
            inv = 1.0 / jnp.sum(p, axis=-1, keepdims=True)
            pnh = p * inv
            pf_parts.append(pnh)
            pn_parts.append(pnh.astype(jnp.bfloat16))
        attn_ref[layer] = ((pf_parts[0] + pf_parts[1])
                           + (pf_parts[2] + pf_parts[3])) * (1.0 / NHEAD)
        pn = jnp.concatenate(pn_parts, axis=2)

        ctx = jax.lax.dot_general(pn, v4, (((2,), (1,)), ((0,), (0,))),
                                  preferred_element_type=jnp.float32)
        attn_out = jnp.dot(ctx.reshape(M, D_MODEL).astype(jnp.bfloat16),
                           wo_ref[layer], preferred_element_type=jnp.float32)

        h1 = _ln(xm + attn_out + bo, g1, be1)
        f = jnp.dot(h1.astype(jnp.bfloat16), w1_ref[layer],
                    preferred_element_type=jnp.float32) + b1_ref[layer]
        f = jnp.maximum(f, 0.0)
        f = jnp.dot(f.astype(jnp.bfloat16), w2_ref[layer],
                    preferred_element_type=jnp.float32) + b2
        xm = _ln(h1 + f, g2, be2)

    y_ref[...] = xm.reshape(bblk, L, D_MODEL)


def kernel(src, begin_offsets, end_offsets, wq, wk, wv, bq, bk, bv,
           wo, bo, w1, b1, w2, b2, g1, be1, g2, be2):
    del begin_offsets
    B = src.shape[1]
    nb = pl.cdiv(B, BBLK)
    scale = math.log2(math.e) / math.sqrt(HEAD_DIM)

    rows = jnp.arange(NHEAD * L)[:, None] // L
    cols = jnp.arange(D_MODEL)[None, :] // HEAD_DIM
    headmask = (rows == cols).astype(jnp.bfloat16)

    wqkv = jnp.concatenate([wq * scale, wk, wv], axis=2).astype(jnp.bfloat16)
    bqkv = jnp.concatenate([bq * scale, bk, bv], axis=2)
    vec = jnp.concatenate([bo, b2, g1, be1, g2, be2], axis=1)

    bias = jnp.where(jnp.arange(L)[None, :] >= end_offsets[:, None],
                     NEG_INF, 0.0).astype(jnp.float32)

    in_specs = [
        pl.BlockSpec((L, BBLK, D_MODEL), lambda b: (0, b, 0)),
        pl.BlockSpec((BBLK, L), lambda b: (b, 0)),
        pl.BlockSpec((NHEAD * L, D_MODEL), lambda b: (0, 0)),
        pl.BlockSpec((N_LAYERS, D_MODEL, 3 * D_MODEL), lambda b: (0, 0, 0)),
        pl.BlockSpec((N_LAYERS, D_MODEL, D_MODEL), lambda b: (0, 0, 0)),
        pl.BlockSpec((N_LAYERS, D_MODEL, DFF), lambda b: (0, 0, 0)),
        pl.BlockSpec((N_LAYERS, DFF, D_MODEL), lambda b: (0, 0, 0)),
        pl.BlockSpec((N_LAYERS, 1, 3 * D_MODEL), lambda b: (0, 0, 0)),
        pl.BlockSpec((N_LAYERS, 1, DFF), lambda b: (0, 0, 0)),
        pl.BlockSpec((N_LAYERS, 6, D_MODEL), lambda b: (0, 0, 0)),
    ]
    out_specs = [
        pl.BlockSpec((BBLK, L, D_MODEL), lambda b: (b, 0, 0)),
        pl.BlockSpec((N_LAYERS, BBLK, L, L), lambda b: (0, b, 0, 0)),
    ]
    out_shape = [
        jax.ShapeDtypeStruct((B, L, D_MODEL), jnp.float32),
        jax.ShapeDtypeStruct((N_LAYERS, B, L, L), jnp.float32),
    ]

    y, attn = pl.pallas_call(
        _encoder_kernel,
        grid=(nb,),
        in_specs=in_specs,
        out_specs=out_specs,
        out_shape=out_shape,
        compiler_params=pltpu.CompilerParams(
            dimension_semantics=("parallel",)),
    )(src, bias, headmask, wqkv, wo.astype(jnp.bfloat16),
      w1.astype(jnp.bfloat16), w2.astype(jnp.bfloat16), bqkv, b1, vec)
    return jnp.transpose(y, (1, 0, 2)), attn


# speedup vs baseline: 5.6382x; 1.0811x over previous
import math

import jax
import jax.numpy as jnp
from jax.experimental import pallas as pl
from jax.experimental.pallas import tpu as pltpu

L = 128
D_MODEL = 32
NHEAD = 4
HEAD_DIM = D_MODEL // NHEAD
DFF = 64
N_LAYERS = 2
EPS = 1e-5
NEG_INF = -1e30
BBLK = 16


def _ln(x, g, b):
    mu = jnp.mean(x, axis=-1, keepdims=True)
    var = jnp.mean((x - mu) ** 2, axis=-1, keepdims=True)
    return (x - mu) * jax.lax.rsqrt(var + EPS) * g + b


def _encoder_kernel(x_ref, bias_ref, hm_ref, wqkv_ref, wo_ref, w1_ref, w2_ref,
                    bqkv_ref, b1_ref, vec_ref, y_ref, attn_ref):
    bblk = x_ref.shape[0]
    M = bblk * L

    headmask = hm_ref[...]

    xm = x_ref[...].reshape(M, D_MODEL)
    bias = bias_ref[...]

    for layer in range(N_LAYERS):
        vec = vec_ref[layer]
        bo, b2 = vec[0:1], vec[1:2]
        g1, be1 = vec[2:3], vec[3:4]
        g2, be2 = vec[4:5], vec[5:6]

        xb = xm.astype(jnp.bfloat16)
        wqkv = wqkv_ref[layer]
        bqkv = bqkv_ref[layer]
        q = (jnp.dot(xb, wqkv[:, :D_MODEL],
                     preferred_element_type=jnp.float32)
             + bqkv[:, :D_MODEL]).astype(jnp.bfloat16).reshape(
                 bblk, L, D_MODEL)
        k = (jnp.dot(xb, wqkv[:, D_MODEL:2 * D_MODEL],
                     preferred_element_type=jnp.float32)
             + bqkv[:, D_MODEL:2 * D_MODEL]).astype(jnp.bfloat16).reshape(
                 bblk, L, D_MODEL)
        v = (jnp.dot(xb, wqkv[:, 2 * D_MODEL:],
                     preferred_element_type=jnp.float32)
             + bqkv[:, 2 * D_MODEL:]).astype(jnp.bfloat16).reshape(
                 bblk, L, D_MODEL)

        k4 = jnp.concatenate([k, k, k, k], axis=1) * headmask
        v4 = jnp.concatenate([v, v, v, v], axis=1) * headmask

        s = jax.lax.dot_general(q, k4, (((2,), (2,)), ((0,), (0,))),
                                preferred_element_type=jnp.float32)

        pn_parts = []
        pf_parts = []
        for h in range(NHEAD):
            sh = s[:, :, h * L:(h + 1) * L] + bias[:, None, :]
            m = jnp.max(sh, axis=-1, keepdims=True)
            p = jnp.exp2(sh - m)
            inv = 1.0 / jnp.sum(p, axis=-1, keepdims=True)
            pnh = p * inv
            pf_parts.append(pnh)
            pn_parts.append(pnh.astype(jnp.bfloat16))
        attn_ref[layer] = ((pf_parts[0] + pf_parts[1])
                           + (pf_parts[2] + pf_parts[3])) * (1.0 / NHEAD)
        pn = jnp.concatenate(pn_parts, axis=2)

        ctx = jax.lax.dot_general(pn, v4, (((2,), (1,)), ((0,), (0,))),
                                  preferred_element_type=jnp.float32)
        attn_out = jnp.dot(ctx.reshape(M, D_MODEL).astype(jnp.bfloat16),
                           wo_ref[layer], preferred_element_type=jnp.float32)

        h1 = _ln(xm + attn_out + bo, g1, be1)
        f = jnp.dot(h1.astype(jnp.bfloat16), w1_ref[layer],
                    preferred_element_type=jnp.float32) + b1_ref[layer]
        f = jnp.maximum(f, 0.0)
        f = jnp.dot(f.astype(jnp.bfloat16), w2_ref[layer],
                    preferred_element_type=jnp.float32) + b2
        xm = _ln(h1 + f, g2, be2)

    y_ref[...] = xm.reshape(bblk, L, D_MODEL)


def kernel(src, begin_offsets, end_offsets, wq, wk, wv, bq, bk, bv,
           wo, bo, w1, b1, w2, b2, g1, be1, g2, be2):
    del begin_offsets
    B = src.shape[1]
    nb = pl.cdiv(B, BBLK)
    scale = math.log2(math.e) / math.sqrt(HEAD_DIM)

    rows = jnp.arange(NHEAD * L)[:, None] // L
    cols = jnp.arange(D_MODEL)[None, :] // HEAD_DIM
    headmask = (rows == cols).astype(jnp.bfloat16)

    wqkv = jnp.concatenate([wq * scale, wk, wv], axis=2).astype(jnp.bfloat16)
    bqkv = jnp.concatenate([bq * scale, bk, bv], axis=2)
    vec = jnp.concatenate([bo, b2, g1, be1, g2, be2], axis=1)

    bias = jnp.where(jnp.arange(L)[None, :] >= end_offsets[:, None],
                     NEG_INF, 0.0).astype(jnp.float32)

    in_specs = [
        pl.BlockSpec((BBLK, L, D_MODEL), lambda b: (b, 0, 0)),
        pl.BlockSpec((BBLK, L), lambda b: (b, 0)),
        pl.BlockSpec((NHEAD * L, D_MODEL), lambda b: (0, 0)),
        pl.BlockSpec((N_LAYERS, D_MODEL, 3 * D_MODEL), lambda b: (0, 0, 0)),
        pl.BlockSpec((N_LAYERS, D_MODEL, D_MODEL), lambda b: (0, 0, 0)),
        pl.BlockSpec((N_LAYERS, D_MODEL, DFF), lambda b: (0, 0, 0)),
        pl.BlockSpec((N_LAYERS, DFF, D_MODEL), lambda b: (0, 0, 0)),
        pl.BlockSpec((N_LAYERS, 1, 3 * D_MODEL), lambda b: (0, 0, 0)),
        pl.BlockSpec((N_LAYERS, 1, DFF), lambda b: (0, 0, 0)),
        pl.BlockSpec((N_LAYERS, 6, D_MODEL), lambda b: (0, 0, 0)),
    ]
    out_specs = [
        pl.BlockSpec((BBLK, L, D_MODEL), lambda b: (b, 0, 0)),
        pl.BlockSpec((N_LAYERS, BBLK, L, L), lambda b: (0, b, 0, 0)),
    ]
    out_shape = [
        jax.ShapeDtypeStruct((B, L, D_MODEL), jnp.float32),
        jax.ShapeDtypeStruct((N_LAYERS, B, L, L), jnp.float32),
    ]

    y, attn = pl.pallas_call(
        _encoder_kernel,
        grid=(nb,),
        in_specs=in_specs,
        out_specs=out_specs,
        out_shape=out_shape,
        compiler_params=pltpu.CompilerParams(
            dimension_semantics=("parallel",)),
    )(jnp.transpose(src, (1, 0, 2)), bias, headmask, wqkv,
      wo.astype(jnp.bfloat16),
      w1.astype(jnp.bfloat16), w2.astype(jnp.bfloat16), bqkv, b1, vec)
    return jnp.transpose(y, (1, 0, 2)), attn
